# Initial kernel scaffold; baseline (speedup 1.0000x reference)
import jax
import jax.numpy as jnp
from jax.experimental import pallas as pl


def kernel(x, edge_index, batch, W1, b1, W2, b2, Wout, bout):
    raise NotImplementedError("write your pallas kernel here")


# speedup vs baseline: 11.5819x; 11.5819x over previous
import functools

import jax
import jax.numpy as jnp
from jax import lax
from jax.experimental import pallas as pl
from jax.experimental.pallas import tpu as pltpu
from jax.experimental.pallas import tpu_sc as plsc

NN = 10000
NE = 320000
DIN = 128
DHID = 256
NG = 64
DH = 128

_NTILE = 16
_K = 80
_NBLKD = NE // (2 * _NTILE) // _K
_NBLK = NE // _NTILE // _K
_CB = 10
_NCH = _NBLK // _CB
_CBD = 5
_NCHD = _NBLKD // _CBD
_STRIPE = 624
_TAIL = NN - _NTILE * _STRIPE

_mesh2 = plsc.VectorSubcoreMesh(core_axis_name="c", subcore_axis_name="s")
_mesh1 = plsc.VectorSubcoreMesh(core_axis_name="c", subcore_axis_name="s",
                                num_cores=1)


@functools.partial(
    pl.kernel,
    out_type=jax.ShapeDtypeStruct((2 * _NTILE, NN), jnp.float32),
    mesh=_mesh2,
    compiler_params=pltpu.CompilerParams(needs_layout_passes=False),
    scratch_types=[
        pltpu.VMEM((_CBD, _K), jnp.int32),
        pltpu.VMEM((NN,), jnp.float32),
    ],
)
def _deg_kernel(dst_hbm, out_hbm, dbuf, hist):
    c = lax.axis_index("c")
    s = lax.axis_index("s")
    w = c * _NTILE + s
    zeros16 = jnp.zeros((16,), jnp.float32)
    ones16 = jnp.full((16,), 1.0, jnp.float32)

    def _z(i, _):
        hist[pl.ds(i * 16, 16)] = zeros16
        return 0

    lax.fori_loop(0, NN // 16, _z, 0)

    def _chunk(ci, _):
        pltpu.sync_copy(dst_hbm.at[w, ci], dbuf)
        for j in range(_CBD):
            for k in range(_K // 16):
                idx = dbuf[j, pl.ds(k * 16, 16)]
                plsc.addupdate_scatter(hist, [idx], ones16)
        return 0

    lax.fori_loop(0, _NCHD, _chunk, 0)
    pltpu.sync_copy(hist, out_hbm.at[w])


@functools.partial(
    pl.kernel,
    out_type=jax.ShapeDtypeStruct((NN, DH), jnp.float32),
    mesh=_mesh1,
    scratch_types=[
        pltpu.VMEM((_CB, _K), jnp.int32),
        pltpu.VMEM((_CB, _K), jnp.int32),
        pltpu.VMEM((_K, DH), jnp.float32),
        pltpu.VMEM((_K, DH), jnp.float32),
        pltpu.VMEM((48, DH), jnp.float32),
        pltpu.VMEM_SHARED((NN, DH), jnp.float32),
        pltpu.SemaphoreType.DMA,
        pltpu.SemaphoreType.DMA,
    ],
)
def _agg_kernel(xs_hbm, src_hbm, dst_hbm, out_hbm, sbuf, dbuf, gb0, gb1, zbuf,
                acc, gsem0, gsem1):
    s = lax.axis_index("s")

    def _zero_row(i, _):
        for k in range(DH // 16):
            zbuf[i, pl.ds(k * 16, 16)] = jnp.zeros((16,), jnp.float32)
        return 0

    lax.fori_loop(0, 48, _zero_row, 0)
    for k in range(13):
        pltpu.sync_copy(zbuf, acc.at[pl.ds(s * _STRIPE + k * 48, 48)])

    @pl.when(s == _NTILE - 1)
    def _():
        pltpu.sync_copy(zbuf.at[pl.ds(0, _TAIL)],
                        acc.at[pl.ds(_NTILE * _STRIPE, _TAIL)])

    plsc.subcore_barrier()

    def _chunk(ci, _):
        pltpu.sync_copy(src_hbm.at[s, ci], sbuf)
        pltpu.sync_copy(dst_hbm.at[s, ci], dbuf)
        bufs = ((gb0, gsem0), (gb1, gsem1))
        d = pltpu.async_copy(xs_hbm.at[sbuf.at[0]], gb0, gsem0)
        for j in range(_CB):
            gb, _sem = bufs[j % 2]
            d.wait()
            if j + 1 < _CB:
                nb, nsem = bufs[(j + 1) % 2]
                d = pltpu.async_copy(xs_hbm.at[sbuf.at[j + 1]], nb, nsem)
            pltpu.sync_copy(gb, acc.at[dbuf.at[j]], add=True)
        return 0

    lax.fori_loop(0, _NCH, _chunk, 0)
    plsc.subcore_barrier()
    pltpu.sync_copy(acc.at[pl.ds(s * _STRIPE, _STRIPE)],
                    out_hbm.at[pl.ds(s * _STRIPE, _STRIPE)])

    @pl.when(s == _NTILE - 1)
    def _():
        pltpu.sync_copy(acc.at[pl.ds(_NTILE * _STRIPE, _TAIL)],
                        out_hbm.at[pl.ds(_NTILE * _STRIPE, _TAIL)])


def _tc1_body(hist_ref, x_ref, dinv_ref, xs1_ref):
    ones_col = jnp.ones((2 * _NTILE, 1), jnp.float32)
    deg = lax.dot_general(hist_ref[...], ones_col, (((0,), (0,)), ((), ())),
                          preferred_element_type=jnp.float32) + 1.0
    dinv = lax.rsqrt(deg)
    dinv_ref[...] = dinv
    xs1_ref[...] = dinv * x_ref[...]


def _tc2_body(S1_ref, xs1_ref, dinv_ref, W1_ref, b1_ref, xs2a_ref, xs2b_ref):
    pre = (S1_ref[...] + xs1_ref[...]) * dinv_ref[...]
    h1 = jnp.dot(pre, W1_ref[...], preferred_element_type=jnp.float32)
    h1 = jnp.maximum(h1 + b1_ref[...], 0.0)
    hs = h1 * dinv_ref[...]
    xs2a_ref[...] = hs[:, :DH]
    xs2b_ref[...] = hs[:, DH:]


def _tc3_body(S2a_ref, S2b_ref, xs2a_ref, xs2b_ref, dinv_ref, batch_ref,
              W2_ref, b2_ref, Wout_ref, bout_ref, out_ref):
    pre = jnp.concatenate(
        [S2a_ref[...] + xs2a_ref[...], S2b_ref[...] + xs2b_ref[...]], axis=1)
    pre = pre * dinv_ref[...]
    h2 = jnp.dot(pre, W2_ref[...], preferred_element_type=jnp.float32)
    h2 = jnp.maximum(h2 + b2_ref[...], 0.0)
    onehot = (batch_ref[...] ==
              lax.broadcasted_iota(jnp.int32, (1, NG), 1)).astype(jnp.float32)
    sums = lax.dot_general(onehot, h2, (((0,), (0,)), ((), ())),
                           preferred_element_type=jnp.float32)
    counts = jnp.sum(onehot, axis=0)[:, None]
    g = sums / jnp.maximum(counts, 1.0)
    out_ref[...] = (jnp.dot(g, Wout_ref[...],
                            preferred_element_type=jnp.float32) + bout_ref[...])


_VSPEC = pl.BlockSpec(memory_space=pltpu.VMEM)


def _tc_call(body, n_in, out_shapes):
    return pl.pallas_call(
        body,
        in_specs=[_VSPEC] * n_in,
        out_specs=[_VSPEC] * len(out_shapes) if len(out_shapes) > 1 else _VSPEC,
        out_shape=(out_shapes if len(out_shapes) > 1 else out_shapes[0]),
    )


def kernel(x, edge_index, batch, W1, b1, W2, b2, Wout, bout):
    src = edge_index[0]
    dst = edge_index[1]
    srcT = src.reshape(_NTILE, _NCH, _CB, _K)
    dstT = dst.reshape(_NTILE, _NCH, _CB, _K)

    deg2 = _deg_kernel(dst.reshape(2 * _NTILE, _NCHD, _CBD, _K))

    f32 = jnp.float32
    dinv, xs1 = _tc_call(
        _tc1_body, 2,
        [jax.ShapeDtypeStruct((NN, 1), f32),
         jax.ShapeDtypeStruct((NN, DIN), f32)])(deg2, x)

    S1 = _agg_kernel(xs1, srcT, dstT)

    xs2a, xs2b = _tc_call(
        _tc2_body, 5,
        [jax.ShapeDtypeStruct((NN, DH), f32),
         jax.ShapeDtypeStruct((NN, DH), f32)])(
            S1, xs1, dinv, W1, b1.reshape(1, DHID))

    S2a = _agg_kernel(xs2a, srcT, dstT)
    S2b = _agg_kernel(xs2b, srcT, dstT)

    out = _tc_call(
        _tc3_body, 10,
        [jax.ShapeDtypeStruct((NG, 1), f32)])(
            S2a, S2b, xs2a, xs2b, dinv,
            batch.reshape(NN, 1), W2, b2.reshape(1, DHID),
            Wout, bout.reshape(1, 1))
    return out


# speedup vs baseline: 12.8797x; 1.1120x over previous
import functools

import jax
import jax.numpy as jnp
from jax import lax
from jax.experimental import pallas as pl
from jax.experimental.pallas import tpu as pltpu
from jax.experimental.pallas import tpu_sc as plsc

NN = 10000
NE = 320000
DIN = 128
DHID = 256
NG = 64
DH = 128

_NTILE = 16
_K = 100
_KD = 80
_NBLKD = NE // (2 * _NTILE) // _KD
_NBLK = NE // _NTILE // _K
_CB = 10
_NCH = _NBLK // _CB
_CBD = 5
_NCHD = _NBLKD // _CBD
_STRIPE = 624
_TAIL = NN - _NTILE * _STRIPE

_mesh2 = plsc.VectorSubcoreMesh(core_axis_name="c", subcore_axis_name="s")
_mesh1 = plsc.VectorSubcoreMesh(core_axis_name="c", subcore_axis_name="s",
                                num_cores=1)


@functools.partial(
    pl.kernel,
    out_type=jax.ShapeDtypeStruct((2 * _NTILE, NN), jnp.float32),
    mesh=_mesh2,
    compiler_params=pltpu.CompilerParams(needs_layout_passes=False),
    scratch_types=[
        pltpu.VMEM((_CBD, _KD), jnp.int32),
        pltpu.VMEM((NN,), jnp.float32),
    ],
)
def _deg_kernel(dst_hbm, out_hbm, dbuf, hist):
    c = lax.axis_index("c")
    s = lax.axis_index("s")
    w = c * _NTILE + s
    zeros16 = jnp.zeros((16,), jnp.float32)
    ones16 = jnp.full((16,), 1.0, jnp.float32)

    def _z(i, _):
        hist[pl.ds(i * 16, 16)] = zeros16
        return 0

    lax.fori_loop(0, NN // 16, _z, 0)

    def _chunk(ci, _):
        pltpu.sync_copy(dst_hbm.at[w, ci], dbuf)
        for j in range(_CBD):
            for k in range(_KD // 16):
                idx = dbuf[j, pl.ds(k * 16, 16)]
                plsc.addupdate_scatter(hist, [idx], ones16)
        return 0

    lax.fori_loop(0, _NCHD, _chunk, 0)
    pltpu.sync_copy(hist, out_hbm.at[w])


@functools.partial(
    pl.kernel,
    out_type=jax.ShapeDtypeStruct((NN, DH), jnp.float32),
    mesh=_mesh1,
    scratch_types=[
        pltpu.VMEM((_CB, _K), jnp.int32),
        pltpu.VMEM((_CB, _K), jnp.int32),
        pltpu.VMEM((_K, DH), jnp.float32),
        pltpu.VMEM((_K, DH), jnp.float32),
        pltpu.VMEM((48, DH), jnp.float32),
        pltpu.VMEM_SHARED((NN, DH), jnp.float32),
        pltpu.SemaphoreType.DMA,
        pltpu.SemaphoreType.DMA,
    ],
)
def _agg_kernel(xs_hbm, src_hbm, dst_hbm, out_hbm, sbuf, dbuf, gb0, gb1, zbuf,
                acc, gsem0, gsem1):
    s = lax.axis_index("s")

    def _zero_row(i, _):
        for k in range(DH // 16):
            zbuf[i, pl.ds(k * 16, 16)] = jnp.zeros((16,), jnp.float32)
        return 0

    lax.fori_loop(0, 48, _zero_row, 0)
    for k in range(13):
        pltpu.sync_copy(zbuf, acc.at[pl.ds(s * _STRIPE + k * 48, 48)])

    @pl.when(s == _NTILE - 1)
    def _():
        pltpu.sync_copy(zbuf.at[pl.ds(0, _TAIL)],
                        acc.at[pl.ds(_NTILE * _STRIPE, _TAIL)])

    plsc.subcore_barrier()

    def _chunk(ci, _):
        pltpu.sync_copy(src_hbm.at[s, ci], sbuf)
        pltpu.sync_copy(dst_hbm.at[s, ci], dbuf)
        bufs = ((gb0, gsem0), (gb1, gsem1))
        d = pltpu.async_copy(xs_hbm.at[sbuf.at[0]], gb0, gsem0)
        for j in range(_CB):
            gb, _sem = bufs[j % 2]
            d.wait()
            if j + 1 < _CB:
                nb, nsem = bufs[(j + 1) % 2]
                d = pltpu.async_copy(xs_hbm.at[sbuf.at[j + 1]], nb, nsem)
            pltpu.sync_copy(gb, acc.at[dbuf.at[j]], add=True)
        return 0

    lax.fori_loop(0, _NCH, _chunk, 0)
    plsc.subcore_barrier()
    pltpu.sync_copy(acc.at[pl.ds(s * _STRIPE, _STRIPE)],
                    out_hbm.at[pl.ds(s * _STRIPE, _STRIPE)])

    @pl.when(s == _NTILE - 1)
    def _():
        pltpu.sync_copy(acc.at[pl.ds(_NTILE * _STRIPE, _TAIL)],
                        out_hbm.at[pl.ds(_NTILE * _STRIPE, _TAIL)])


def _tc1_body(hist_ref, x_ref, dinv_ref, xs1_ref):
    ones_col = jnp.ones((2 * _NTILE, 1), jnp.float32)
    deg = lax.dot_general(hist_ref[...], ones_col, (((0,), (0,)), ((), ())),
                          preferred_element_type=jnp.float32) + 1.0
    dinv = lax.rsqrt(deg)
    dinv_ref[...] = dinv
    xs1_ref[...] = dinv * x_ref[...]


def _tc2_body(S1_ref, xs1_ref, dinv_ref, W1_ref, b1_ref, xs2a_ref, xs2b_ref):
    pre = (S1_ref[...] + xs1_ref[...]) * dinv_ref[...]
    h1 = jnp.dot(pre, W1_ref[...], preferred_element_type=jnp.float32)
    h1 = jnp.maximum(h1 + b1_ref[...], 0.0)
    hs = h1 * dinv_ref[...]
    xs2a_ref[...] = hs[:, :DH]
    xs2b_ref[...] = hs[:, DH:]


def _tc3_body(S2a_ref, S2b_ref, xs2a_ref, xs2b_ref, dinv_ref, batch_ref,
              W2_ref, b2_ref, Wout_ref, bout_ref, out_ref):
    pre = jnp.concatenate(
        [S2a_ref[...] + xs2a_ref[...], S2b_ref[...] + xs2b_ref[...]], axis=1)
    pre = pre * dinv_ref[...]
    h2 = jnp.dot(pre, W2_ref[...], preferred_element_type=jnp.float32)
    h2 = jnp.maximum(h2 + b2_ref[...], 0.0)
    onehot = (batch_ref[...] ==
              lax.broadcasted_iota(jnp.int32, (1, NG), 1)).astype(jnp.float32)
    sums = lax.dot_general(onehot, h2, (((0,), (0,)), ((), ())),
                           preferred_element_type=jnp.float32)
    counts = jnp.sum(onehot, axis=0)[:, None]
    g = sums / jnp.maximum(counts, 1.0)
    out_ref[...] = (jnp.dot(g, Wout_ref[...],
                            preferred_element_type=jnp.float32) + bout_ref[...])


_VSPEC = pl.BlockSpec(memory_space=pltpu.VMEM)


def _tc_call(body, n_in, out_shapes):
    return pl.pallas_call(
        body,
        in_specs=[_VSPEC] * n_in,
        out_specs=[_VSPEC] * len(out_shapes) if len(out_shapes) > 1 else _VSPEC,
        out_shape=(out_shapes if len(out_shapes) > 1 else out_shapes[0]),
    )


def kernel(x, edge_index, batch, W1, b1, W2, b2, Wout, bout):
    src = edge_index[0]
    dst = edge_index[1]
    srcT = src.reshape(_NTILE, _NCH, _CB, _K)
    dstT = dst.reshape(_NTILE, _NCH, _CB, _K)

    deg2 = _deg_kernel(dst.reshape(2 * _NTILE, _NCHD, _CBD, _KD))

    f32 = jnp.float32
    dinv, xs1 = _tc_call(
        _tc1_body, 2,
        [jax.ShapeDtypeStruct((NN, 1), f32),
         jax.ShapeDtypeStruct((NN, DIN), f32)])(deg2, x)

    S1 = _agg_kernel(xs1, srcT, dstT)

    xs2a, xs2b = _tc_call(
        _tc2_body, 5,
        [jax.ShapeDtypeStruct((NN, DH), f32),
         jax.ShapeDtypeStruct((NN, DH), f32)])(
            S1, xs1, dinv, W1, b1.reshape(1, DHID))

    S2a = _agg_kernel(xs2a, srcT, dstT)
    S2b = _agg_kernel(xs2b, srcT, dstT)

    out = _tc_call(
        _tc3_body, 10,
        [jax.ShapeDtypeStruct((NG, 1), f32)])(
            S2a, S2b, xs2a, xs2b, dinv,
            batch.reshape(NN, 1), W2, b2.reshape(1, DHID),
            Wout, bout.reshape(1, 1))
    return out


# speedup vs baseline: 16.1799x; 1.2562x over previous
import functools

import jax
import jax.numpy as jnp
from jax import lax
from jax.experimental import pallas as pl
from jax.experimental.pallas import tpu as pltpu
from jax.experimental.pallas import tpu_sc as plsc

NN = 10000
NE = 320000
DIN = 128
DHID = 256
NG = 64
DH = 128

_NTILE = 16
_K = 100
_KD = 80
_NBLKD = NE // (2 * _NTILE) // _KD
_NBLK = NE // _NTILE // _K
_CB = 10
_NCH = _NBLK // _CB
_CBD = 5
_NCHD = _NBLKD // _CBD
_STRIPE = 624
_TAIL = NN - _NTILE * _STRIPE

_mesh2 = plsc.VectorSubcoreMesh(core_axis_name="c", subcore_axis_name="s")
_mesh1 = plsc.VectorSubcoreMesh(core_axis_name="c", subcore_axis_name="s",
                                num_cores=1)


@functools.partial(
    pl.kernel,
    out_type=jax.ShapeDtypeStruct((2 * _NTILE, NN), jnp.float32),
    mesh=_mesh2,
    compiler_params=pltpu.CompilerParams(needs_layout_passes=False),
    scratch_types=[
        pltpu.VMEM((_CBD, _KD), jnp.int32),
        pltpu.VMEM((NN,), jnp.float32),
    ],
)
def _deg_kernel(dst_hbm, out_hbm, dbuf, hist):
    c = lax.axis_index("c")
    s = lax.axis_index("s")
    w = c * _NTILE + s
    zeros16 = jnp.zeros((16,), jnp.float32)
    ones16 = jnp.full((16,), 1.0, jnp.float32)

    def _z(i, _):
        hist[pl.ds(i * 16, 16)] = zeros16
        return 0

    lax.fori_loop(0, NN // 16, _z, 0)

    def _chunk(ci, _):
        pltpu.sync_copy(dst_hbm.at[w, ci], dbuf)
        for j in range(_CBD):
            for k in range(_KD // 16):
                idx = dbuf[j, pl.ds(k * 16, 16)]
                plsc.addupdate_scatter(hist, [idx], ones16)
        return 0

    lax.fori_loop(0, _NCHD, _chunk, 0)
    pltpu.sync_copy(hist, out_hbm.at[w])


@functools.partial(
    pl.kernel,
    out_type=jax.ShapeDtypeStruct((NN, DH), jnp.float32),
    mesh=_mesh1,
    scratch_types=[
        pltpu.VMEM((_CB, _K), jnp.int32),
        pltpu.VMEM((_CB, _K), jnp.int32),
        pltpu.VMEM((_K, DH), jnp.float32),
        pltpu.VMEM((_K, DH), jnp.float32),
        pltpu.VMEM((_K, DH), jnp.float32),
        pltpu.VMEM_SHARED((NN, DH), jnp.float32),
        pltpu.SemaphoreType.DMA,
        pltpu.SemaphoreType.DMA,
        pltpu.SemaphoreType.DMA,
        pltpu.SemaphoreType.DMA,
        pltpu.SemaphoreType.DMA,
        pltpu.SemaphoreType.DMA,
    ],
)
def _agg_kernel(xs_hbm, src_hbm, dst_hbm, out_hbm, sbuf, dbuf, gb0, gb1, gb2,
                acc, gsem0, gsem1, gsem2, ssem0, ssem1, ssem2):
    s = lax.axis_index("s")

    def _zero_row(i, _):
        for k in range(DH // 16):
            gb0[i, pl.ds(k * 16, 16)] = jnp.zeros((16,), jnp.float32)
        return 0

    lax.fori_loop(0, _K, _zero_row, 0)
    for k in range(6):
        pltpu.sync_copy(gb0.at[pl.ds(0, 96)],
                        acc.at[pl.ds(s * _STRIPE + k * 96, 96)])
    pltpu.sync_copy(gb0.at[pl.ds(0, 48)],
                    acc.at[pl.ds(s * _STRIPE + 576, 48)])

    @pl.when(s == _NTILE - 1)
    def _():
        pltpu.sync_copy(gb0.at[pl.ds(0, _TAIL)],
                        acc.at[pl.ds(_NTILE * _STRIPE, _TAIL)])

    plsc.subcore_barrier()

    gbs = (gb0, gb1, gb2)
    gsems = (gsem0, gsem1, gsem2)
    ssems = (ssem0, ssem1, ssem2)

    def _chunk(ci, _):
        pltpu.sync_copy(src_hbm.at[s, ci], sbuf)
        pltpu.sync_copy(dst_hbm.at[s, ci], dbuf)
        dg = [None, None, None]
        ds = [None, None, None]
        dg[0] = pltpu.async_copy(xs_hbm.at[sbuf.at[0]], gb0, gsem0)
        dg[1] = pltpu.async_copy(xs_hbm.at[sbuf.at[1]], gb1, gsem1)
        for j in range(_CB):
            b = j % 3
            if j + 2 < _CB:
                pb = (j + 2) % 3
                if ds[pb] is not None:
                    ds[pb].wait()
                dg[pb] = pltpu.async_copy(
                    xs_hbm.at[sbuf.at[j + 2]], gbs[pb], gsems[pb])
            dg[b].wait()
            ds[b] = pltpu.async_copy(gbs[b], acc.at[dbuf.at[j]], ssems[b],
                                     add=True)
        for b in range(3):
            if ds[b] is not None:
                ds[b].wait()
        return 0

    lax.fori_loop(0, _NCH, _chunk, 0)
    plsc.subcore_barrier()
    pltpu.sync_copy(acc.at[pl.ds(s * _STRIPE, _STRIPE)],
                    out_hbm.at[pl.ds(s * _STRIPE, _STRIPE)])

    @pl.when(s == _NTILE - 1)
    def _():
        pltpu.sync_copy(acc.at[pl.ds(_NTILE * _STRIPE, _TAIL)],
                        out_hbm.at[pl.ds(_NTILE * _STRIPE, _TAIL)])


def _tc1_body(hist_ref, x_ref, dinv_ref, xs1_ref):
    ones_col = jnp.ones((2 * _NTILE, 1), jnp.float32)
    deg = lax.dot_general(hist_ref[...], ones_col, (((0,), (0,)), ((), ())),
                          preferred_element_type=jnp.float32) + 1.0
    dinv = lax.rsqrt(deg)
    dinv_ref[...] = dinv
    xs1_ref[...] = dinv * x_ref[...]


def _tc2_body(S1_ref, xs1_ref, dinv_ref, W1_ref, b1_ref, xs2a_ref, xs2b_ref):
    pre = (S1_ref[...] + xs1_ref[...]) * dinv_ref[...]
    h1 = jnp.dot(pre, W1_ref[...], preferred_element_type=jnp.float32)
    h1 = jnp.maximum(h1 + b1_ref[...], 0.0)
    hs = h1 * dinv_ref[...]
    xs2a_ref[...] = hs[:, :DH]
    xs2b_ref[...] = hs[:, DH:]


def _tc3_body(S2a_ref, S2b_ref, xs2a_ref, xs2b_ref, dinv_ref, batch_ref,
              W2_ref, b2_ref, Wout_ref, bout_ref, out_ref):
    pre = jnp.concatenate(
        [S2a_ref[...] + xs2a_ref[...], S2b_ref[...] + xs2b_ref[...]], axis=1)
    pre = pre * dinv_ref[...]
    h2 = jnp.dot(pre, W2_ref[...], preferred_element_type=jnp.float32)
    h2 = jnp.maximum(h2 + b2_ref[...], 0.0)
    onehot = (batch_ref[...] ==
              lax.broadcasted_iota(jnp.int32, (1, NG), 1)).astype(jnp.float32)
    sums = lax.dot_general(onehot, h2, (((0,), (0,)), ((), ())),
                           preferred_element_type=jnp.float32)
    counts = jnp.sum(onehot, axis=0)[:, None]
    g = sums / jnp.maximum(counts, 1.0)
    out_ref[...] = (jnp.dot(g, Wout_ref[...],
                            preferred_element_type=jnp.float32) + bout_ref[...])


_VSPEC = pl.BlockSpec(memory_space=pltpu.VMEM)


def _tc_call(body, n_in, out_shapes):
    return pl.pallas_call(
        body,
        in_specs=[_VSPEC] * n_in,
        out_specs=[_VSPEC] * len(out_shapes) if len(out_shapes) > 1 else _VSPEC,
        out_shape=(out_shapes if len(out_shapes) > 1 else out_shapes[0]),
    )


def kernel(x, edge_index, batch, W1, b1, W2, b2, Wout, bout):
    src = edge_index[0]
    dst = edge_index[1]
    srcT = src.reshape(_NTILE, _NCH, _CB, _K)
    dstT = dst.reshape(_NTILE, _NCH, _CB, _K)

    deg2 = _deg_kernel(dst.reshape(2 * _NTILE, _NCHD, _CBD, _KD))

    f32 = jnp.float32
    dinv, xs1 = _tc_call(
        _tc1_body, 2,
        [jax.ShapeDtypeStruct((NN, 1), f32),
         jax.ShapeDtypeStruct((NN, DIN), f32)])(deg2, x)

    S1 = _agg_kernel(xs1, srcT, dstT)

    xs2a, xs2b = _tc_call(
        _tc2_body, 5,
        [jax.ShapeDtypeStruct((NN, DH), f32),
         jax.ShapeDtypeStruct((NN, DH), f32)])(
            S1, xs1, dinv, W1, b1.reshape(1, DHID))

    S2a = _agg_kernel(xs2a, srcT, dstT)
    S2b = _agg_kernel(xs2b, srcT, dstT)

    out = _tc_call(
        _tc3_body, 10,
        [jax.ShapeDtypeStruct((NG, 1), f32)])(
            S2a, S2b, xs2a, xs2b, dinv,
            batch.reshape(NN, 1), W2, b2.reshape(1, DHID),
            Wout, bout.reshape(1, 1))
    return out
